# baseline (speedup 1.0000x reference)
import functools

import jax
import jax.numpy as jnp
from jax import lax
from jax.experimental import pallas as pl
from jax.experimental.pallas import tpu as pltpu
from jax.experimental.pallas import tpu_sc as plsc

NUM_ENTITIES = 1000000
NUM_RELATIONS = 1000
DIM = 64
BATCH = 16384

_info = plsc.get_sparse_core_info()
NC, NS, L = _info.num_cores, _info.num_subcores, _info.num_lanes
NW = NC * NS
B_PER_W = BATCH // NW
CHUNK = 64
N_CHUNKS = B_PER_W // CHUNK
NBUF = 4


def _row_buf_types():
  return [
      pltpu.VMEM((CHUNK, 2 * DIM), jnp.float32),
      pltpu.VMEM((CHUNK, 2 * DIM), jnp.float32),
      pltpu.VMEM((CHUNK, DIM), jnp.float32),
      pltpu.VMEM((CHUNK, DIM), jnp.float32),
      pltpu.VMEM((CHUNK,), jnp.float32),
      pltpu.SemaphoreType.DMA,
      pltpu.SemaphoreType.DMA,
  ]


def _make_kernel():
  mesh = plsc.VectorSubcoreMesh(core_axis_name="c", subcore_axis_name="s")

  buf_types = []
  for _ in range(NBUF):
    buf_types += _row_buf_types()

  @functools.partial(
      pl.kernel,
      mesh=mesh,
      compiler_params=pltpu.CompilerParams(
          needs_layout_passes=False, use_tc_tiling_on_sc=False),
      out_type=jax.ShapeDtypeStruct((BATCH,), jnp.float32),
      scratch_types=[
          pltpu.VMEM((N_CHUNKS, CHUNK), jnp.int32),
          pltpu.VMEM((N_CHUNKS, CHUNK), jnp.int32),
          pltpu.VMEM((N_CHUNKS, CHUNK), jnp.int32),
      ] + [pltpu.SemaphoreType.DMA] * N_CHUNKS + buf_types,
  )
  def scores(enc_h, h_h, r_h, t_h, rre_h, rim_h, out_h,
             idxh_v, idxt_v, idxr_v, *rest):
    isems = rest[:N_CHUNKS]
    nb = len(_row_buf_types())
    bufs = tuple(rest[N_CHUNKS + k * nb: N_CHUNKS + (k + 1) * nb]
                 for k in range(NBUF))
    wid = lax.axis_index("s") * NC + lax.axis_index("c")
    lanes = lax.iota(jnp.int32, L)

    icps = []
    for j in range(N_CHUNKS):
      base = wid * B_PER_W + j * CHUNK
      icps.append((
          pltpu.async_copy(h_h.at[pl.ds(base, CHUNK)], idxh_v.at[j], isems[j]),
          pltpu.async_copy(t_h.at[pl.ds(base, CHUNK)], idxt_v.at[j], isems[j]),
          pltpu.async_copy(r_h.at[pl.ds(base, CHUNK)], idxr_v.at[j], isems[j]),
      ))

    def fire(j):
      eh, et, rr, ri, _, sem, _ = bufs[j % NBUF]
      for cp in icps[j]:
        cp.wait()
      return (pltpu.async_copy(enc_h.at[idxh_v.at[j]], eh, sem),
              pltpu.async_copy(enc_h.at[idxt_v.at[j]], et, sem),
              pltpu.async_copy(rre_h.at[idxr_v.at[j]], rr, sem),
              pltpu.async_copy(rim_h.at[idxr_v.at[j]], ri, sem))

    cps = [fire(j) for j in range(NBUF - 1)] + [None] * (N_CHUNKS - NBUF + 1)
    ocps = [None] * NBUF
    for j in range(N_CHUNKS):
      eh_v, et_v, rr_v, ri_v, out_v, _, osem = bufs[j % NBUF]
      if j + NBUF - 1 < N_CHUNKS:
        cps[j + NBUF - 1] = fire(j + NBUF - 1)
      for cp in cps[j]:
        cp.wait()
      if ocps[j % NBUF] is not None:
        ocps[j % NBUF].wait()

      @plsc.parallel_loop(0, CHUNK // L)
      def group(g):
        gbase = g * L

        @plsc.parallel_loop(0, L, unroll=4, carry=jnp.zeros((L,), jnp.float32))
        def out_vec(tt, ovec):
          i = gbase + tt
          acc = jnp.zeros((L,), jnp.float32)
          for c in range(DIM // L):
            lo = c * L
            ehr = eh_v[i, pl.ds(lo, L)]
            ehi = eh_v[i, pl.ds(DIM + lo, L)]
            etr = et_v[i, pl.ds(lo, L)]
            eti = et_v[i, pl.ds(DIM + lo, L)]
            rre = rr_v[i, pl.ds(lo, L)]
            rim = ri_v[i, pl.ds(lo, L)]
            acc = acc + ehr * (rre * etr + rim * eti) + ehi * (rre * eti - rim * etr)
          s = jnp.sum(acc)
          return ovec + jnp.where(lanes == tt, s, jnp.float32(0))

        out_v[pl.ds(gbase, L)] = out_vec

      ocps[j % NBUF] = pltpu.async_copy(
          out_v, out_h.at[pl.ds(wid * B_PER_W + j * CHUNK, CHUNK)], osem)

    for ocp in ocps:
      if ocp is not None:
        ocp.wait()

  return scores


_scores = _make_kernel()


@jax.jit
def kernel(enc, h, r, t, rel_re, rel_im):
  h = h.astype(jnp.int32)
  r = r.astype(jnp.int32)
  t = t.astype(jnp.int32)
  return _scores(enc, h, r, t, rel_re, rel_im)


# speedup vs baseline: 1.1485x; 1.1485x over previous
import functools

import jax
import jax.numpy as jnp
from jax import lax
from jax.experimental import pallas as pl
from jax.experimental.pallas import tpu as pltpu
from jax.experimental.pallas import tpu_sc as plsc

NUM_ENTITIES = 1000000
NUM_RELATIONS = 1000
DIM = 64
BATCH = 16384

_info = plsc.get_sparse_core_info()
NC, NS, L = _info.num_cores, _info.num_subcores, _info.num_lanes
NW = NC * NS
B_PER_W = BATCH // NW
CHUNK = 128
N_CHUNKS = B_PER_W // CHUNK


def _row_buf_types():
  return [
      pltpu.VMEM((CHUNK, 2 * DIM), jnp.float32),
      pltpu.VMEM((CHUNK, 2 * DIM), jnp.float32),
      pltpu.VMEM((CHUNK, DIM), jnp.bfloat16),
      pltpu.VMEM((CHUNK, DIM), jnp.bfloat16),
      pltpu.VMEM((CHUNK,), jnp.float32),
      pltpu.SemaphoreType.DMA,
  ]


def _make_kernel():
  mesh = plsc.VectorSubcoreMesh(core_axis_name="c", subcore_axis_name="s")

  @functools.partial(
      pl.kernel,
      mesh=mesh,
      compiler_params=pltpu.CompilerParams(
          needs_layout_passes=False, use_tc_tiling_on_sc=False),
      out_type=jax.ShapeDtypeStruct((BATCH,), jnp.float32),
      scratch_types=[
          pltpu.VMEM((N_CHUNKS, CHUNK), jnp.int32),
          pltpu.VMEM((N_CHUNKS, CHUNK), jnp.int32),
          pltpu.VMEM((N_CHUNKS, CHUNK), jnp.int32),
          pltpu.SemaphoreType.DMA,
          pltpu.SemaphoreType.DMA,
          pltpu.SemaphoreType.DMA,
          pltpu.SemaphoreType.DMA,
          pltpu.SemaphoreType.DMA,
          pltpu.SemaphoreType.DMA,
      ] + _row_buf_types() + _row_buf_types(),
  )
  def scores(enc_h, h_h, r_h, t_h, rre_h, rim_h, out_h,
             idxh_v, idxt_v, idxr_v, si0, si1, si2, si3, so0, so1, *scratch):
    bufs = (scratch[:6], scratch[6:])
    isems = (si0, si1, si2, si3)
    osems = (so0, so1)
    wid = lax.axis_index("s") * NC + lax.axis_index("c")
    lanes = lax.iota(jnp.int32, L)

    icps = []
    for j in range(N_CHUNKS):
      base = wid * B_PER_W + j * CHUNK
      icps.append((
          pltpu.async_copy(h_h.at[pl.ds(base, CHUNK)], idxh_v.at[j], isems[j]),
          pltpu.async_copy(t_h.at[pl.ds(base, CHUNK)], idxt_v.at[j], isems[j]),
          pltpu.async_copy(r_h.at[pl.ds(base, CHUNK)], idxr_v.at[j], isems[j]),
      ))

    def fire(j):
      eh, et, rr, ri, _, sem = bufs[j % 2]
      for cp in icps[j]:
        cp.wait()
      return (pltpu.async_copy(enc_h.at[idxh_v.at[j]], eh, sem),
              pltpu.async_copy(enc_h.at[idxt_v.at[j]], et, sem),
              pltpu.async_copy(rre_h.at[idxr_v.at[j]], rr, sem),
              pltpu.async_copy(rim_h.at[idxr_v.at[j]], ri, sem))

    cps = fire(0)
    ocps = [None, None]
    for j in range(N_CHUNKS):
      eh_v, et_v, rr_v, ri_v, out_v, _ = bufs[j % 2]
      nxt = fire(j + 1) if j + 1 < N_CHUNKS else None
      for cp in cps:
        cp.wait()
      if ocps[j % 2] is not None:
        ocps[j % 2].wait()

      @plsc.parallel_loop(0, CHUNK // L)
      def group(g):
        gbase = g * L

        @plsc.parallel_loop(0, L, unroll=8, carry=jnp.zeros((L,), jnp.float32))
        def out_vec(tt, ovec):
          i = gbase + tt
          acc = jnp.zeros((L,), jnp.float32)
          for c2 in range(DIM // (2 * L)):
            rr_ab = plsc.unpack(rr_v[i, pl.ds(c2 * 2 * L, 2 * L)],
                                format=plsc.PackFormat.INTERLEAVED)
            ri_ab = plsc.unpack(ri_v[i, pl.ds(c2 * 2 * L, 2 * L)],
                                format=plsc.PackFormat.INTERLEAVED)
            for k in range(2):
              lo = c2 * 2 * L + k * L
              rre = rr_ab[k]
              rim = ri_ab[k]
              ehr = eh_v[i, pl.ds(lo, L)]
              ehi = eh_v[i, pl.ds(DIM + lo, L)]
              etr = et_v[i, pl.ds(lo, L)]
              eti = et_v[i, pl.ds(DIM + lo, L)]
              acc = acc + ehr * (rre * etr + rim * eti) + ehi * (rre * eti - rim * etr)
          s = jnp.sum(acc)
          return ovec + jnp.where(lanes == tt, s, jnp.float32(0))

        out_v[pl.ds(gbase, L)] = out_vec

      ocps[j % 2] = pltpu.async_copy(
          out_v, out_h.at[pl.ds(wid * B_PER_W + j * CHUNK, CHUNK)], osems[j % 2])
      cps = nxt

    for ocp in ocps:
      if ocp is not None:
        ocp.wait()

  return scores


_scores = _make_kernel()


_PERM = []
for _b in range(DIM // 32):
  for _l in range(16):
    _PERM.extend([_b * 32 + _l, _b * 32 + 16 + _l])
_PERM_ARR = tuple(_PERM)


@jax.jit
def kernel(enc, h, r, t, rel_re, rel_im):
  h = h.astype(jnp.int32)
  r = r.astype(jnp.int32)
  t = t.astype(jnp.int32)
  perm = jnp.array(_PERM_ARR, dtype=jnp.int32)
  rel_re_p = rel_re[:, perm].astype(jnp.bfloat16)
  rel_im_p = rel_im[:, perm].astype(jnp.bfloat16)
  return _scores(enc, h, r, t, rel_re_p, rel_im_p)
